# baseline (speedup 1.0000x reference)
import jax
import jax.numpy as jnp
from jax.experimental import pallas as pl

E = 8
T = 2048
H = 2048
TB = 128


def _fused_body(eps_ref, scale_ref, *rest):
    act_refs = rest[:E]
    token_ref, resid_ref, nw_ref, hs_ref, outres_ref = rest[E:]
    acc = token_ref[...] + resid_ref[...]
    scol = scale_ref[...].T
    for e in range(E):
        acc = acc + act_refs[e][0] * scol[:, e][:, None]
    outres_ref[...] = acc
    var = jnp.mean(acc * acc, axis=-1, keepdims=True)
    hs_ref[...] = acc * jax.lax.rsqrt(var + eps_ref[0]) * nw_ref[...]


def kernel(residual, norm_weight, device_num_experts, scale_input,
           active_experts_token_input, token_input, eps):
    del device_num_experts
    eps_arr = jnp.asarray(eps, dtype=jnp.float32).reshape(1)
    nw = norm_weight.reshape(1, H)

    def mk_act_spec(e):
        return pl.BlockSpec((1, TB, H), lambda i, _e=e: (_e, i, 0))

    return pl.pallas_call(
        _fused_body,
        grid=(T // TB,),
        in_specs=[
            pl.BlockSpec((1,), lambda i: (0,)),
            pl.BlockSpec((E, TB), lambda i: (0, i)),
            *[mk_act_spec(e) for e in range(E)],
            pl.BlockSpec((TB, H), lambda i: (i, 0)),
            pl.BlockSpec((TB, H), lambda i: (i, 0)),
            pl.BlockSpec((1, H), lambda i: (0, 0)),
        ],
        out_specs=[
            pl.BlockSpec((TB, H), lambda i: (i, 0)),
            pl.BlockSpec((TB, H), lambda i: (i, 0)),
        ],
        out_shape=[
            jax.ShapeDtypeStruct((T, H), jnp.float32),
            jax.ShapeDtypeStruct((T, H), jnp.float32),
        ],
    )(eps_arr, scale_input,
      *([active_experts_token_input] * E),
      token_input, residual, nw)


# speedup vs baseline: 1.0067x; 1.0067x over previous
import jax
import jax.numpy as jnp
from jax.experimental import pallas as pl

E = 8
T = 2048
H = 2048
TB = 128


def _fused_body(eps_ref, scale_ref, active_ref, token_ref, resid_ref, nw_ref,
                hs_ref, outres_ref):
    acc = token_ref[...] + resid_ref[...]
    scol = scale_ref[...].T
    for e in range(E):
        acc = acc + active_ref[e] * scol[:, e][:, None]
    outres_ref[...] = acc
    var = jnp.mean(acc * acc, axis=-1, keepdims=True)
    hs_ref[...] = acc * jax.lax.rsqrt(var + eps_ref[0]) * nw_ref[...]


def kernel(residual, norm_weight, device_num_experts, scale_input,
           active_experts_token_input, token_input, eps):
    del device_num_experts
    eps_arr = jnp.asarray(eps, dtype=jnp.float32).reshape(1)
    nw = norm_weight.reshape(1, H)

    return pl.pallas_call(
        _fused_body,
        grid=(T // TB,),
        in_specs=[
            pl.BlockSpec((1,), lambda i: (0,)),
            pl.BlockSpec((E, TB), lambda i: (0, i)),
            pl.BlockSpec((E, TB, H), lambda i: (0, i, 0)),
            pl.BlockSpec((TB, H), lambda i: (i, 0)),
            pl.BlockSpec((TB, H), lambda i: (i, 0)),
            pl.BlockSpec((1, H), lambda i: (0, 0)),
        ],
        out_specs=[
            pl.BlockSpec((TB, H), lambda i: (i, 0)),
            pl.BlockSpec((TB, H), lambda i: (i, 0)),
        ],
        out_shape=[
            jax.ShapeDtypeStruct((T, H), jnp.float32),
            jax.ShapeDtypeStruct((T, H), jnp.float32),
        ],
    )(eps_arr, scale_input, active_experts_token_input, token_input,
      residual, nw)
